# Initial kernel scaffold; baseline (speedup 1.0000x reference)
import jax
import jax.numpy as jnp
from jax.experimental import pallas as pl


def kernel(user_emb, item_emb, edge_index, edge_vals):
    raise NotImplementedError("write your pallas kernel here")


# speedup vs baseline: 5.0702x; 5.0702x over previous
import functools

import jax
import jax.numpy as jnp
from jax import lax
from jax.experimental import pallas as pl
from jax.experimental.pallas import tpu as pltpu
from jax.experimental.pallas import tpu_sc as plsc

USER_N = 50000
ITEM_N = 50000
N = USER_N + ITEM_N
E = 1600000
D = 32

NC = 2
NS = 16
L = 16

HALF = N // NC
DUMMY = HALF
ACC_ROWS = HALF + 8

BLK = 512
SCAT = 128
NCH = BLK // SCAT
NBLK = E // BLK

WB = 128
ROWS_PER_TILE = 3128
LAST_ROWS = HALF - (NS - 1) * ROWS_PER_TILE
WB_FULL = 24
WB_TAIL = ROWS_PER_TILE - WB_FULL * WB
WB_TAIL_LAST = LAST_ROWS - WB_FULL * WB

_mesh = plsc.VectorSubcoreMesh(
    core_axis_name="c", subcore_axis_name="s", num_cores=NC, num_subcores=NS)


def _make_layer(scale):
    @functools.partial(
        pl.kernel,
        out_type=(
            jax.ShapeDtypeStruct((N, D), jnp.float32),
            jax.ShapeDtypeStruct((N, D), jnp.float32),
        ),
        mesh=_mesh,
        compiler_params=pltpu.CompilerParams(use_tc_tiling_on_sc=False),
        scratch_types=[
            pltpu.VMEM_SHARED((ACC_ROWS, D), jnp.float32),
            pltpu.VMEM((BLK,), jnp.int32),
            pltpu.VMEM((BLK,), jnp.int32),
            pltpu.VMEM((BLK,), jnp.float32),
            pltpu.VMEM((NCH, SCAT), jnp.int32),
            pltpu.VMEM((BLK, D), jnp.float32),
            pltpu.VMEM((WB, D), jnp.float32),
            pltpu.VMEM((WB, D), jnp.float32),
            pltpu.SemaphoreType.DMA,
            pltpu.SemaphoreType.DMA,
        ],
    )
    def layer(ego, src, dst, vals, sum_in, ego_out, sum_out,
              acc, src_v, dst_v, vals_v, dstl_v, rows_v, wb_v, sum_v,
              gsem, ssem):
        c = lax.axis_index("c")
        s = lax.axis_index("s")
        half_base = c * HALF
        iota = lax.broadcasted_iota(jnp.int32, (L,), 0)
        zero16 = jnp.zeros((L,), jnp.float32)

        def _zrow(r, _):
            wb_v[r, pl.ds(0, L)] = zero16
            wb_v[r, pl.ds(L, L)] = zero16
            return 0
        lax.fori_loop(0, WB, _zrow, 0)
        my_row0 = s * ROWS_PER_TILE
        for k in range(WB_FULL):
            pltpu.sync_copy(wb_v, acc.at[pl.ds(my_row0 + k * WB, WB), :])

        @pl.when(s < NS - 1)
        def _():
            pltpu.sync_copy(wb_v.at[pl.ds(0, WB_TAIL), :],
                            acc.at[pl.ds(my_row0 + WB_FULL * WB, WB_TAIL), :])

        @pl.when(s == NS - 1)
        def _():
            pltpu.sync_copy(
                wb_v.at[pl.ds(0, WB_TAIL_LAST), :],
                acc.at[pl.ds(my_row0 + WB_FULL * WB, WB_TAIL_LAST), :])
        @pl.when(s == 0)
        def _():
            pltpu.sync_copy(wb_v.at[pl.ds(0, ACC_ROWS - HALF), :],
                            acc.at[pl.ds(HALF, ACC_ROWS - HALF), :])
        plsc.subcore_barrier()

        nblk = jnp.where(s < NBLK % NS, NBLK // NS + 1, NBLK // NS)

        def blk_body(i, _):
            base = (i * NS + s) * BLK
            pltpu.sync_copy(src.at[pl.ds(base, BLK)], src_v)
            pltpu.sync_copy(dst.at[pl.ds(base, BLK)], dst_v)
            pltpu.sync_copy(vals.at[pl.ds(base, BLK)], vals_v)
            descs = []
            for j in range(NCH):
                descs.append(pltpu.async_copy(
                    ego.at[src_v.at[pl.ds(j * SCAT, SCAT)]],
                    rows_v.at[pl.ds(j * SCAT, SCAT), :], gsem))
            for dsc in descs:
                dsc.wait()

            def grp_body(g, _):
                e0 = g * L
                d16 = dst_v[pl.ds(e0, L)]
                lcl = d16 - half_base
                ok = (lcl >= 0) & (lcl < HALF)
                lcl = jnp.where(ok, lcl, DUMMY)
                dstl_v[g // (SCAT // L), pl.ds((g % (SCAT // L)) * L, L)] = lcl
                vv = vals_v[pl.ds(e0, L)]
                for k in range(L):
                    e = e0 + k
                    v = vv[k]
                    rows_v[e, pl.ds(0, L)] = rows_v[e, pl.ds(0, L)] * v
                    rows_v[e, pl.ds(L, L)] = rows_v[e, pl.ds(L, L)] * v
                return 0
            lax.fori_loop(0, BLK // L, grp_body, 0)

            sdescs = []
            for j in range(NCH):
                sdescs.append(pltpu.async_copy(
                    rows_v.at[pl.ds(j * SCAT, SCAT), :],
                    acc.at[dstl_v.at[j]], ssem, add=True))
            for dsc in sdescs:
                dsc.wait()
            return 0
        lax.fori_loop(0, nblk, blk_body, 0)
        plsc.subcore_barrier()

        def wb_chunk(row_off, nrows):
            g_off = half_base + row_off
            pltpu.sync_copy(acc.at[pl.ds(row_off, nrows), :],
                            wb_v.at[pl.ds(0, nrows), :])
            pltpu.sync_copy(sum_in.at[pl.ds(g_off, nrows), :],
                            sum_v.at[pl.ds(0, nrows), :])

            def srow(r, _):
                for h in range(D // L):
                    a = sum_v[r, pl.ds(h * L, L)] + wb_v[r, pl.ds(h * L, L)]
                    sum_v[r, pl.ds(h * L, L)] = a * scale
                return 0
            lax.fori_loop(0, nrows, srow, 0)
            pltpu.sync_copy(sum_v.at[pl.ds(0, nrows), :],
                            sum_out.at[pl.ds(g_off, nrows), :])
            pltpu.sync_copy(wb_v.at[pl.ds(0, nrows), :],
                            ego_out.at[pl.ds(g_off, nrows), :])

        for k in range(WB_FULL):
            wb_chunk(my_row0 + k * WB, WB)

        @pl.when(s < NS - 1)
        def _():
            wb_chunk(my_row0 + WB_FULL * WB, WB_TAIL)

        @pl.when(s == NS - 1)
        def _():
            wb_chunk(my_row0 + WB_FULL * WB, WB_TAIL_LAST)

    return layer


_layer_mid = _make_layer(1.0)
_layer_last = _make_layer(0.25)


def kernel(user_emb, item_emb, edge_index, edge_vals):
    ego = jnp.concatenate([user_emb, item_emb], axis=0)
    src = edge_index[0]
    dst = edge_index[1]
    e1, s1 = _layer_mid(ego, src, dst, edge_vals, ego)
    e2, s2 = _layer_mid(e1, src, dst, edge_vals, s1)
    _, s3 = _layer_last(e2, src, dst, edge_vals, s2)
    return (s3[:USER_N], s3[USER_N:])


# speedup vs baseline: 5.2233x; 1.0302x over previous
import functools

import jax
import jax.numpy as jnp
from jax import lax
from jax.experimental import pallas as pl
from jax.experimental.pallas import tpu as pltpu
from jax.experimental.pallas import tpu_sc as plsc

USER_N = 50000
ITEM_N = 50000
N = USER_N + ITEM_N
E = 1600000
D = 32

NC = 2
NS = 16
L = 16

HALF = N // NC
ACC_ROWS = HALF + 8

BLK = 512
SCAT = 128
NCH = BLK // SCAT
NBLK = E // BLK

WB = 128
ROWS_PER_TILE = 3128
LAST_ROWS = HALF - (NS - 1) * ROWS_PER_TILE
WB_FULL = 24
WB_TAIL = ROWS_PER_TILE - WB_FULL * WB
WB_TAIL_LAST = LAST_ROWS - WB_FULL * WB

_mesh = plsc.VectorSubcoreMesh(
    core_axis_name="c", subcore_axis_name="s", num_cores=NC, num_subcores=NS)


def _make_layer(scale):
    @functools.partial(
        pl.kernel,
        out_type=(
            jax.ShapeDtypeStruct((N, D), jnp.float32),
            jax.ShapeDtypeStruct((N, D), jnp.float32),
        ),
        mesh=_mesh,
        compiler_params=pltpu.CompilerParams(use_tc_tiling_on_sc=False),
        scratch_types=[
            pltpu.VMEM_SHARED((ACC_ROWS, D), jnp.float32),
            pltpu.VMEM((BLK,), jnp.int32),
            pltpu.VMEM((BLK,), jnp.int32),
            pltpu.VMEM((BLK,), jnp.float32),
            pltpu.VMEM((NCH, SCAT), jnp.int32),
            pltpu.VMEM((BLK, D), jnp.float32),
            pltpu.VMEM((WB, D), jnp.float32),
            pltpu.VMEM((WB, D), jnp.float32),
            pltpu.SemaphoreType.DMA,
            pltpu.SemaphoreType.DMA,
        ],
    )
    def layer(ego, src, dst, vals, sum_in, ego_out, sum_out,
              acc, src_v, dst_v, vals_v, dstl_v, rows_v, wb_v, sum_v,
              gsem, ssem):
        c = lax.axis_index("c")
        s = lax.axis_index("s")
        half_base = c * HALF
        iota = lax.broadcasted_iota(jnp.int32, (L,), 0)
        zero16 = jnp.zeros((L,), jnp.float32)

        def _zrow(r, _):
            wb_v[r, pl.ds(0, L)] = zero16
            wb_v[r, pl.ds(L, L)] = zero16
            return 0
        lax.fori_loop(0, WB, _zrow, 0)
        my_row0 = s * ROWS_PER_TILE
        for k in range(WB_FULL):
            pltpu.sync_copy(wb_v, acc.at[pl.ds(my_row0 + k * WB, WB), :])

        @pl.when(s < NS - 1)
        def _():
            pltpu.sync_copy(wb_v.at[pl.ds(0, WB_TAIL), :],
                            acc.at[pl.ds(my_row0 + WB_FULL * WB, WB_TAIL), :])

        @pl.when(s == NS - 1)
        def _():
            pltpu.sync_copy(
                wb_v.at[pl.ds(0, WB_TAIL_LAST), :],
                acc.at[pl.ds(my_row0 + WB_FULL * WB, WB_TAIL_LAST), :])
        plsc.subcore_barrier()

        nblk = jnp.where(s < NBLK % NS, NBLK // NS + 1, NBLK // NS)

        def blk_body(i, _):
            base = (i * NS + s) * BLK
            pltpu.sync_copy(src.at[pl.ds(base, BLK)], src_v)
            pltpu.sync_copy(dst.at[pl.ds(base, BLK)], dst_v)
            pltpu.sync_copy(vals.at[pl.ds(base, BLK)], vals_v)
            descs = []
            for j in range(NCH):
                descs.append(pltpu.async_copy(
                    ego.at[src_v.at[pl.ds(j * SCAT, SCAT)]],
                    rows_v.at[pl.ds(j * SCAT, SCAT), :], gsem))
            for dsc in descs:
                dsc.wait()

            def grp_body(g, _):
                e0 = g * L
                d16 = dst_v[pl.ds(e0, L)]
                lcl = d16 - half_base
                lcl = jnp.where(lcl >= HALF, lcl - HALF,
                                jnp.where(lcl < 0, lcl + HALF, lcl))
                dstl_v[g // (SCAT // L), pl.ds((g % (SCAT // L)) * L, L)] = lcl
                ok = (d16 - half_base >= 0) & (d16 - half_base < HALF)
                vv = jnp.where(ok, vals_v[pl.ds(e0, L)], 0.0)
                for k in range(L):
                    e = e0 + k
                    v = vv[k]
                    rows_v[e, pl.ds(0, L)] = rows_v[e, pl.ds(0, L)] * v
                    rows_v[e, pl.ds(L, L)] = rows_v[e, pl.ds(L, L)] * v
                return 0
            lax.fori_loop(0, BLK // L, grp_body, 0)

            sdescs = []
            for j in range(NCH):
                sdescs.append(pltpu.async_copy(
                    rows_v.at[pl.ds(j * SCAT, SCAT), :],
                    acc.at[dstl_v.at[j]], ssem, add=True))
            for dsc in sdescs:
                dsc.wait()
            return 0
        lax.fori_loop(0, nblk, blk_body, 0)
        plsc.subcore_barrier()

        def wb_chunk(row_off, nrows):
            g_off = half_base + row_off
            pltpu.sync_copy(acc.at[pl.ds(row_off, nrows), :],
                            wb_v.at[pl.ds(0, nrows), :])
            pltpu.sync_copy(sum_in.at[pl.ds(g_off, nrows), :],
                            sum_v.at[pl.ds(0, nrows), :])

            def srow(r, _):
                for h in range(D // L):
                    a = sum_v[r, pl.ds(h * L, L)] + wb_v[r, pl.ds(h * L, L)]
                    sum_v[r, pl.ds(h * L, L)] = a * scale
                return 0
            lax.fori_loop(0, nrows, srow, 0)
            pltpu.sync_copy(sum_v.at[pl.ds(0, nrows), :],
                            sum_out.at[pl.ds(g_off, nrows), :])
            pltpu.sync_copy(wb_v.at[pl.ds(0, nrows), :],
                            ego_out.at[pl.ds(g_off, nrows), :])

        for k in range(WB_FULL):
            wb_chunk(my_row0 + k * WB, WB)

        @pl.when(s < NS - 1)
        def _():
            wb_chunk(my_row0 + WB_FULL * WB, WB_TAIL)

        @pl.when(s == NS - 1)
        def _():
            wb_chunk(my_row0 + WB_FULL * WB, WB_TAIL_LAST)

    return layer


_layer_mid = _make_layer(1.0)
_layer_last = _make_layer(0.25)


def kernel(user_emb, item_emb, edge_index, edge_vals):
    ego = jnp.concatenate([user_emb, item_emb], axis=0)
    src = edge_index[0]
    dst = edge_index[1]
    e1, s1 = _layer_mid(ego, src, dst, edge_vals, ego)
    e2, s2 = _layer_mid(e1, src, dst, edge_vals, s1)
    _, s3 = _layer_last(e2, src, dst, edge_vals, s2)
    return (s3[:USER_N], s3[USER_N:])


# speedup vs baseline: 9.0578x; 1.7341x over previous
import functools

import jax
import jax.numpy as jnp
from jax import lax
from jax.experimental import pallas as pl
from jax.experimental.pallas import tpu as pltpu
from jax.experimental.pallas import tpu_sc as plsc

USER_N = 50000
ITEM_N = 50000
N = USER_N + ITEM_N
E = 1600000
D = 32

NC = 2
NS = 16
L = 16

HALF = N // NC
ACC_ROWS = HALF + 8

BLK = 256
SCAT = 128
NCH = BLK // SCAT
NB = E // BLK
NSLOT = 3

WB = 256
ROWS_PER_TILE = 3128
LAST_ROWS = HALF - (NS - 1) * ROWS_PER_TILE
WB_FULL = 12
WB_TAIL = ROWS_PER_TILE - WB_FULL * WB
WB_TAIL_LAST = LAST_ROWS - WB_FULL * WB

_mesh = plsc.VectorSubcoreMesh(
    core_axis_name="c", subcore_axis_name="s", num_cores=NC, num_subcores=NS)


def _make_layer(scale):
    @functools.partial(
        pl.kernel,
        out_type=(
            jax.ShapeDtypeStruct((N, D), jnp.float32),
            jax.ShapeDtypeStruct((N, D), jnp.float32),
        ),
        mesh=_mesh,
        compiler_params=pltpu.CompilerParams(use_tc_tiling_on_sc=False),
        scratch_types=(
            [pltpu.VMEM_SHARED((ACC_ROWS, D), jnp.float32)]
            + [pltpu.VMEM((2, BLK), jnp.int32) for _ in range(NSLOT)]
            + [pltpu.VMEM((BLK,), jnp.float32) for _ in range(NSLOT)]
            + [pltpu.VMEM((BLK, D), jnp.float32) for _ in range(NSLOT)]
            + [pltpu.VMEM((NCH, SCAT), jnp.int32) for _ in range(NSLOT)]
            + [pltpu.SemaphoreType.DMA for _ in range(3 * NSLOT)]
        ),
    )
    def layer(ego, edata, vals, sum_in, ego_out, sum_out, acc, *scr):
        edv = scr[0:NSLOT]
        valv = scr[NSLOT:2 * NSLOT]
        rows = scr[2 * NSLOT:3 * NSLOT]
        dstl = scr[3 * NSLOT:4 * NSLOT]
        isem = scr[4 * NSLOT:5 * NSLOT]
        gsem = scr[5 * NSLOT:6 * NSLOT]
        ssem = scr[6 * NSLOT:7 * NSLOT]

        c = lax.axis_index("c")
        s = lax.axis_index("s")
        half_base = c * HALF
        zero16 = jnp.zeros((L,), jnp.float32)

        def _zrow(r, _):
            rows[0][r, pl.ds(0, L)] = zero16
            rows[0][r, pl.ds(L, L)] = zero16
            return 0
        lax.fori_loop(0, WB, _zrow, 0)
        my_row0 = s * ROWS_PER_TILE
        for k in range(WB_FULL):
            pltpu.sync_copy(rows[0], acc.at[pl.ds(my_row0 + k * WB, WB), :])

        @pl.when(s < NS - 1)
        def _():
            pltpu.sync_copy(rows[0].at[pl.ds(0, WB_TAIL), :],
                            acc.at[pl.ds(my_row0 + WB_FULL * WB, WB_TAIL), :])

        @pl.when(s == NS - 1)
        def _():
            pltpu.sync_copy(
                rows[0].at[pl.ds(0, WB_TAIL_LAST), :],
                acc.at[pl.ds(my_row0 + WB_FULL * WB, WB_TAIL_LAST), :])
        plsc.subcore_barrier()

        n_i = jnp.int32(NB // NS) + (s < NB % NS)

        def fire_idx(sl, i):
            b = i * NS + s
            pltpu.async_copy(edata.at[b], edv[sl], isem[sl])
            pltpu.async_copy(vals.at[pl.ds(b * BLK, BLK)], valv[sl], isem[sl])

        def wait_idx(sl, i):
            b = i * NS + s
            pltpu.make_async_copy(edata.at[b], edv[sl], isem[sl]).wait()
            pltpu.make_async_copy(
                vals.at[pl.ds(b * BLK, BLK)], valv[sl], isem[sl]).wait()

        def fire_gather(sl):
            for j in range(NCH):
                pltpu.async_copy(
                    ego.at[edv[sl].at[0, pl.ds(j * SCAT, SCAT)]],
                    rows[sl].at[pl.ds(j * SCAT, SCAT), :], gsem[sl])

        def wait_gather(sl):
            for j in range(NCH):
                pltpu.make_async_copy(
                    ego.at[edv[sl].at[0, pl.ds(j * SCAT, SCAT)]],
                    rows[sl].at[pl.ds(j * SCAT, SCAT), :], gsem[sl]).wait()

        def fire_scatter(sl):
            for j in range(NCH):
                pltpu.async_copy(
                    rows[sl].at[pl.ds(j * SCAT, SCAT), :],
                    acc.at[dstl[sl].at[j]], ssem[sl], add=True)

        def drain_scatter(sl):
            for j in range(NCH):
                pltpu.make_async_copy(
                    rows[sl].at[pl.ds(j * SCAT, SCAT), :],
                    acc.at[dstl[sl].at[j]], ssem[sl]).wait()

        def compute(sl):
            def grp_body(g, _):
                e0 = g * L
                d16 = edv[sl][1, pl.ds(e0, L)]
                lcl = d16 - half_base
                ok = (lcl >= 0) & (lcl < HALF)
                lcl = jnp.where(lcl >= HALF, lcl - HALF,
                                jnp.where(lcl < 0, lcl + HALF, lcl))
                dstl[sl][g // (SCAT // L),
                         pl.ds((g % (SCAT // L)) * L, L)] = lcl
                vv = jnp.where(ok, valv[sl][pl.ds(e0, L)], 0.0)
                for k in range(L):
                    e = e0 + k
                    v = vv[k]
                    rows[sl][e, pl.ds(0, L)] = rows[sl][e, pl.ds(0, L)] * v
                    rows[sl][e, pl.ds(L, L)] = rows[sl][e, pl.ds(L, L)] * v
                return 0
            lax.fori_loop(0, BLK // L, grp_body, 0)

        fire_idx(0, jnp.int32(0))
        wait_idx(0, jnp.int32(0))
        fire_gather(0)
        fire_idx(1, jnp.int32(1))

        def pipe_iter(i, sl):
            @pl.when(i >= 2)
            def _():
                drain_scatter((sl + 1) % NSLOT)

            @pl.when(i + 1 < n_i)
            def _():
                wait_idx((sl + 1) % NSLOT, i + 1)
                fire_gather((sl + 1) % NSLOT)
            wait_gather(sl)

            @pl.when(i + 2 < n_i)
            def _():
                fire_idx((sl + 2) % NSLOT, i + 2)
            compute(sl)
            fire_scatter(sl)

        def body(i3, _):
            for p in range(NSLOT):
                i = i3 * NSLOT + p

                @pl.when(i < n_i)
                def _():
                    pipe_iter(i, p)
            return 0
        lax.fori_loop(0, (n_i + NSLOT - 1) // NSLOT, body, 0)

        for sl in range(NSLOT):
            @pl.when((((n_i - 1) % NSLOT) == sl) | (((n_i - 2) % NSLOT) == sl))
            def _():
                drain_scatter(sl)
        plsc.subcore_barrier()

        def wb_chunk(row_off, nrows):
            g_off = half_base + row_off
            pltpu.sync_copy(acc.at[pl.ds(row_off, nrows), :],
                            rows[0].at[pl.ds(0, nrows), :])
            pltpu.sync_copy(sum_in.at[pl.ds(g_off, nrows), :],
                            rows[1].at[pl.ds(0, nrows), :])

            def srow(r, _):
                for h in range(D // L):
                    a = (rows[1][r, pl.ds(h * L, L)]
                         + rows[0][r, pl.ds(h * L, L)])
                    rows[1][r, pl.ds(h * L, L)] = a * scale
                return 0
            lax.fori_loop(0, nrows, srow, 0)
            pltpu.sync_copy(rows[1].at[pl.ds(0, nrows), :],
                            sum_out.at[pl.ds(g_off, nrows), :])
            pltpu.sync_copy(rows[0].at[pl.ds(0, nrows), :],
                            ego_out.at[pl.ds(g_off, nrows), :])

        for k in range(WB_FULL):
            wb_chunk(my_row0 + k * WB, WB)

        @pl.when(s < NS - 1)
        def _():
            wb_chunk(my_row0 + WB_FULL * WB, WB_TAIL)

        @pl.when(s == NS - 1)
        def _():
            wb_chunk(my_row0 + WB_FULL * WB, WB_TAIL_LAST)

    return layer


_layer_mid = _make_layer(1.0)
_layer_last = _make_layer(0.25)


def kernel(user_emb, item_emb, edge_index, edge_vals):
    ego = jnp.concatenate([user_emb, item_emb], axis=0)
    edata = edge_index.reshape(2, NB, BLK).transpose(1, 0, 2)
    e1, s1 = _layer_mid(ego, edata, edge_vals, ego)
    e2, s2 = _layer_mid(e1, edata, edge_vals, s1)
    _, s3 = _layer_last(e2, edata, edge_vals, s2)
    return (s3[:USER_N], s3[USER_N:])


# speedup vs baseline: 23.3470x; 2.5776x over previous
import functools

import jax
import jax.numpy as jnp
from jax import lax
from jax.experimental import pallas as pl
from jax.experimental.pallas import tpu as pltpu
from jax.experimental.pallas import tpu_sc as plsc

USER_N = 50000
ITEM_N = 50000
N = USER_N + ITEM_N
E = 1600000
D = 32

NC = 2
NS = 16
L = 16
DH = D // NC

ACC_ROWS = N + 8

BLK = 512
SCAT = 128
NCH = BLK // SCAT
NB = E // BLK
NSLOT = 3

WB = 512
ROWS_PER_TILE = 6256
LAST_ROWS = N - (NS - 1) * ROWS_PER_TILE
WB_FULL = 12
WB_TAIL = ROWS_PER_TILE - WB_FULL * WB
WB_TAIL_LAST = LAST_ROWS - WB_FULL * WB

_mesh = plsc.VectorSubcoreMesh(
    core_axis_name="c", subcore_axis_name="s", num_cores=NC, num_subcores=NS)


def _make_layer(scale):
    @functools.partial(
        pl.kernel,
        out_type=(
            jax.ShapeDtypeStruct((NC, N, DH), jnp.float32),
            jax.ShapeDtypeStruct((NC, N, DH), jnp.float32),
        ),
        mesh=_mesh,
        compiler_params=pltpu.CompilerParams(use_tc_tiling_on_sc=False),
        scratch_types=(
            [pltpu.VMEM_SHARED((ACC_ROWS, DH), jnp.float32)]
            + [pltpu.VMEM((2, BLK), jnp.int32) for _ in range(NSLOT)]
            + [pltpu.VMEM((BLK,), jnp.float32) for _ in range(NSLOT)]
            + [pltpu.VMEM((BLK, DH), jnp.float32) for _ in range(NSLOT)]
            + [pltpu.VMEM((NCH, SCAT), jnp.int32) for _ in range(NSLOT)]
            + [pltpu.SemaphoreType.DMA for _ in range(3 * NSLOT)]
        ),
    )
    def layer(ego, edata, vals, sum_in, ego_out, sum_out, acc, *scr):
        edv = scr[0:NSLOT]
        valv = scr[NSLOT:2 * NSLOT]
        rows = scr[2 * NSLOT:3 * NSLOT]
        dstl = scr[3 * NSLOT:4 * NSLOT]
        isem = scr[4 * NSLOT:5 * NSLOT]
        gsem = scr[5 * NSLOT:6 * NSLOT]
        ssem = scr[6 * NSLOT:7 * NSLOT]

        c = lax.axis_index("c")
        s = lax.axis_index("s")
        zero16 = jnp.zeros((L,), jnp.float32)
        my_ego = ego.at[c]

        def _zrow(r, _):
            rows[0][r, pl.ds(0, L)] = zero16
            return 0
        lax.fori_loop(0, WB, _zrow, 0)
        my_row0 = s * ROWS_PER_TILE
        for k in range(WB_FULL):
            pltpu.sync_copy(rows[0], acc.at[pl.ds(my_row0 + k * WB, WB), :])

        @pl.when(s < NS - 1)
        def _():
            pltpu.sync_copy(rows[0].at[pl.ds(0, WB_TAIL), :],
                            acc.at[pl.ds(my_row0 + WB_FULL * WB, WB_TAIL), :])

        @pl.when(s == NS - 1)
        def _():
            pltpu.sync_copy(
                rows[0].at[pl.ds(0, WB_TAIL_LAST), :],
                acc.at[pl.ds(my_row0 + WB_FULL * WB, WB_TAIL_LAST), :])
        plsc.subcore_barrier()

        n_i = jnp.int32(NB // NS) + (s < NB % NS)

        def fire_idx(sl, i):
            b = i * NS + s
            pltpu.async_copy(edata.at[b], edv[sl], isem[sl])
            pltpu.async_copy(vals.at[pl.ds(b * BLK, BLK)], valv[sl], isem[sl])

        def wait_idx(sl, i):
            b = i * NS + s
            pltpu.make_async_copy(edata.at[b], edv[sl], isem[sl]).wait()
            pltpu.make_async_copy(
                vals.at[pl.ds(b * BLK, BLK)], valv[sl], isem[sl]).wait()

        def fire_gather(sl):
            for j in range(NCH):
                pltpu.async_copy(
                    my_ego.at[edv[sl].at[0, pl.ds(j * SCAT, SCAT)]],
                    rows[sl].at[pl.ds(j * SCAT, SCAT), :], gsem[sl])

        def wait_gather(sl):
            for j in range(NCH):
                pltpu.make_async_copy(
                    my_ego.at[edv[sl].at[0, pl.ds(j * SCAT, SCAT)]],
                    rows[sl].at[pl.ds(j * SCAT, SCAT), :], gsem[sl]).wait()

        def fire_scatter(sl):
            for j in range(NCH):
                pltpu.async_copy(
                    rows[sl].at[pl.ds(j * SCAT, SCAT), :],
                    acc.at[dstl[sl].at[j]], ssem[sl], add=True)

        def drain_scatter(sl):
            for j in range(NCH):
                pltpu.make_async_copy(
                    rows[sl].at[pl.ds(j * SCAT, SCAT), :],
                    acc.at[dstl[sl].at[j]], ssem[sl]).wait()

        def compute(sl):
            def grp_body(g, _):
                e0 = g * L
                d16 = edv[sl][1, pl.ds(e0, L)]
                dstl[sl][g // (SCAT // L),
                         pl.ds((g % (SCAT // L)) * L, L)] = d16
                vv = valv[sl][pl.ds(e0, L)]
                for k in range(L):
                    e = e0 + k
                    v = vv[k]
                    rows[sl][e, pl.ds(0, L)] = rows[sl][e, pl.ds(0, L)] * v
                return 0
            lax.fori_loop(0, BLK // L, grp_body, 0)

        fire_idx(0, jnp.int32(0))
        wait_idx(0, jnp.int32(0))
        fire_gather(0)
        fire_idx(1, jnp.int32(1))

        def pipe_iter(i, sl):
            @pl.when(i >= 2)
            def _():
                drain_scatter((sl + 1) % NSLOT)

            @pl.when(i + 1 < n_i)
            def _():
                wait_idx((sl + 1) % NSLOT, i + 1)
                fire_gather((sl + 1) % NSLOT)
            wait_gather(sl)

            @pl.when(i + 2 < n_i)
            def _():
                fire_idx((sl + 2) % NSLOT, i + 2)
            compute(sl)
            fire_scatter(sl)

        def body(i3, _):
            for p in range(NSLOT):
                i = i3 * NSLOT + p

                @pl.when(i < n_i)
                def _():
                    pipe_iter(i, p)
            return 0
        lax.fori_loop(0, (n_i + NSLOT - 1) // NSLOT, body, 0)

        for sl in range(NSLOT):
            @pl.when((((n_i - 1) % NSLOT) == sl) | (((n_i - 2) % NSLOT) == sl))
            def _():
                drain_scatter(sl)
        plsc.subcore_barrier()

        def wb_chunk(row_off, nrows):
            pltpu.sync_copy(acc.at[pl.ds(row_off, nrows), :],
                            rows[0].at[pl.ds(0, nrows), :])
            pltpu.sync_copy(sum_in.at[c, pl.ds(row_off, nrows), :],
                            rows[1].at[pl.ds(0, nrows), :])

            def srow(r, _):
                a = rows[1][r, pl.ds(0, L)] + rows[0][r, pl.ds(0, L)]
                rows[1][r, pl.ds(0, L)] = a * scale
                return 0
            lax.fori_loop(0, nrows, srow, 0)
            pltpu.sync_copy(rows[1].at[pl.ds(0, nrows), :],
                            sum_out.at[c, pl.ds(row_off, nrows), :])
            pltpu.sync_copy(rows[0].at[pl.ds(0, nrows), :],
                            ego_out.at[c, pl.ds(row_off, nrows), :])

        for k in range(WB_FULL):
            wb_chunk(my_row0 + k * WB, WB)

        @pl.when(s < NS - 1)
        def _():
            wb_chunk(my_row0 + WB_FULL * WB, WB_TAIL)

        @pl.when(s == NS - 1)
        def _():
            wb_chunk(my_row0 + WB_FULL * WB, WB_TAIL_LAST)

    return layer


_layer_mid = _make_layer(1.0)
_layer_last = _make_layer(0.25)


def kernel(user_emb, item_emb, edge_index, edge_vals):
    ego = jnp.concatenate([user_emb, item_emb], axis=0)
    egoh = jnp.stack([ego[:, :DH], ego[:, DH:]], axis=0)
    edata = edge_index.reshape(2, NB, BLK).transpose(1, 0, 2)
    e1, s1 = _layer_mid(egoh, edata, edge_vals, egoh)
    e2, s2 = _layer_mid(e1, edata, edge_vals, s1)
    _, s3 = _layer_last(e2, edata, edge_vals, s2)
    mean = jnp.concatenate([s3[0], s3[1]], axis=1)
    return (mean[:USER_N], mean[USER_N:])


# speedup vs baseline: 24.0557x; 1.0304x over previous
import functools

import jax
import jax.numpy as jnp
from jax import lax
from jax.experimental import pallas as pl
from jax.experimental.pallas import tpu as pltpu
from jax.experimental.pallas import tpu_sc as plsc

USER_N = 50000
ITEM_N = 50000
N = USER_N + ITEM_N
E = 1600000
D = 32

NC = 2
NS = 16
L = 16
DH = D // NC

ACC_ROWS = N + 8

BLK = 256
SCAT = 128
NCH = BLK // SCAT
NB = E // BLK
NSLOT = 6
GLA = 2
ILA = 4

WB = 256
ROWS_PER_TILE = 6256
LAST_ROWS = N - (NS - 1) * ROWS_PER_TILE
WB_FULL = 24
WB_TAIL = ROWS_PER_TILE - WB_FULL * WB
WB_TAIL_LAST = LAST_ROWS - WB_FULL * WB

_mesh = plsc.VectorSubcoreMesh(
    core_axis_name="c", subcore_axis_name="s", num_cores=NC, num_subcores=NS)


def _make_layer(scale):
    @functools.partial(
        pl.kernel,
        out_type=(
            jax.ShapeDtypeStruct((NC, N, DH), jnp.float32),
            jax.ShapeDtypeStruct((NC, N, DH), jnp.float32),
        ),
        mesh=_mesh,
        compiler_params=pltpu.CompilerParams(use_tc_tiling_on_sc=False),
        scratch_types=(
            [pltpu.VMEM_SHARED((ACC_ROWS, DH), jnp.float32)]
            + [pltpu.VMEM((2, BLK), jnp.int32) for _ in range(NSLOT)]
            + [pltpu.VMEM((BLK,), jnp.float32) for _ in range(NSLOT)]
            + [pltpu.VMEM((BLK, DH), jnp.float32) for _ in range(NSLOT)]
            + [pltpu.VMEM((NCH, SCAT), jnp.int32) for _ in range(NSLOT)]
            + [pltpu.SemaphoreType.DMA for _ in range(3 * NSLOT)]
        ),
    )
    def layer(ego, edata, vals, sum_in, ego_out, sum_out, acc, *scr):
        edv = scr[0:NSLOT]
        valv = scr[NSLOT:2 * NSLOT]
        rows = scr[2 * NSLOT:3 * NSLOT]
        dstl = scr[3 * NSLOT:4 * NSLOT]
        isem = scr[4 * NSLOT:5 * NSLOT]
        gsem = scr[5 * NSLOT:6 * NSLOT]
        ssem = scr[6 * NSLOT:7 * NSLOT]

        c = lax.axis_index("c")
        s = lax.axis_index("s")
        zero16 = jnp.zeros((L,), jnp.float32)
        my_ego = ego.at[c]

        def _zrow(r, _):
            rows[0][r, pl.ds(0, L)] = zero16
            return 0
        lax.fori_loop(0, WB, _zrow, 0)
        my_row0 = s * ROWS_PER_TILE
        for k in range(WB_FULL):
            pltpu.sync_copy(rows[0], acc.at[pl.ds(my_row0 + k * WB, WB), :])

        @pl.when(s < NS - 1)
        def _():
            pltpu.sync_copy(rows[0].at[pl.ds(0, WB_TAIL), :],
                            acc.at[pl.ds(my_row0 + WB_FULL * WB, WB_TAIL), :])

        @pl.when(s == NS - 1)
        def _():
            pltpu.sync_copy(
                rows[0].at[pl.ds(0, WB_TAIL_LAST), :],
                acc.at[pl.ds(my_row0 + WB_FULL * WB, WB_TAIL_LAST), :])
        plsc.subcore_barrier()

        n_i = jnp.int32(NB // NS) + (s < NB % NS)

        def fire_idx(sl, i):
            b = i * NS + s
            pltpu.async_copy(edata.at[b], edv[sl], isem[sl])
            pltpu.async_copy(vals.at[pl.ds(b * BLK, BLK)], valv[sl], isem[sl])

        def wait_idx(sl, i):
            b = i * NS + s
            pltpu.make_async_copy(edata.at[b], edv[sl], isem[sl]).wait()
            pltpu.make_async_copy(
                vals.at[pl.ds(b * BLK, BLK)], valv[sl], isem[sl]).wait()

        def fire_gather(sl):
            for j in range(NCH):
                pltpu.async_copy(
                    my_ego.at[edv[sl].at[0, pl.ds(j * SCAT, SCAT)]],
                    rows[sl].at[pl.ds(j * SCAT, SCAT), :], gsem[sl])

        def wait_gather(sl):
            for j in range(NCH):
                pltpu.make_async_copy(
                    my_ego.at[edv[sl].at[0, pl.ds(j * SCAT, SCAT)]],
                    rows[sl].at[pl.ds(j * SCAT, SCAT), :], gsem[sl]).wait()

        def fire_scatter(sl):
            for j in range(NCH):
                pltpu.async_copy(
                    rows[sl].at[pl.ds(j * SCAT, SCAT), :],
                    acc.at[dstl[sl].at[j]], ssem[sl], add=True)

        def drain_scatter(sl):
            for j in range(NCH):
                pltpu.make_async_copy(
                    rows[sl].at[pl.ds(j * SCAT, SCAT), :],
                    acc.at[dstl[sl].at[j]], ssem[sl]).wait()

        def compute(sl):
            def grp_body(g, _):
                e0 = g * L
                d16 = edv[sl][1, pl.ds(e0, L)]
                dstl[sl][g // (SCAT // L),
                         pl.ds((g % (SCAT // L)) * L, L)] = d16
                vv = valv[sl][pl.ds(e0, L)]
                for k in range(L):
                    e = e0 + k
                    v = vv[k]
                    rows[sl][e, pl.ds(0, L)] = rows[sl][e, pl.ds(0, L)] * v
                return 0
            lax.fori_loop(0, BLK // L, grp_body, 0)

        for k in range(ILA):
            fire_idx(k, jnp.int32(k))
        for k in range(GLA):
            wait_idx(k, jnp.int32(k))
            fire_gather(k)

        def pipe_iter(i, sl):
            @pl.when(i >= NSLOT - GLA)
            def _():
                drain_scatter((sl + GLA) % NSLOT)

            @pl.when(i + GLA < n_i)
            def _():
                wait_idx((sl + GLA) % NSLOT, i + GLA)
                fire_gather((sl + GLA) % NSLOT)
            wait_gather(sl)

            @pl.when(i + ILA < n_i)
            def _():
                fire_idx((sl + ILA) % NSLOT, i + ILA)
            compute(sl)
            fire_scatter(sl)

        def body(i6, _):
            for p in range(NSLOT):
                i = i6 * NSLOT + p

                @pl.when(i < n_i)
                def _():
                    pipe_iter(i, p)
            return 0
        lax.fori_loop(0, (n_i + NSLOT - 1) // NSLOT, body, 0)

        for sl in range(NSLOT):
            conds = [((n_i - 1 - k) % NSLOT) == sl for k in range(NSLOT - GLA)]
            cond = conds[0]
            for cd in conds[1:]:
                cond = cond | cd

            @pl.when(cond)
            def _():
                drain_scatter(sl)
        plsc.subcore_barrier()

        def wb_chunk(row_off, nrows):
            pltpu.sync_copy(acc.at[pl.ds(row_off, nrows), :],
                            rows[0].at[pl.ds(0, nrows), :])
            pltpu.sync_copy(sum_in.at[c, pl.ds(row_off, nrows), :],
                            rows[1].at[pl.ds(0, nrows), :])

            def srow(r, _):
                a = rows[1][r, pl.ds(0, L)] + rows[0][r, pl.ds(0, L)]
                rows[1][r, pl.ds(0, L)] = a * scale
                return 0
            lax.fori_loop(0, nrows, srow, 0)
            pltpu.sync_copy(rows[1].at[pl.ds(0, nrows), :],
                            sum_out.at[c, pl.ds(row_off, nrows), :])
            pltpu.sync_copy(rows[0].at[pl.ds(0, nrows), :],
                            ego_out.at[c, pl.ds(row_off, nrows), :])

        for k in range(WB_FULL):
            wb_chunk(my_row0 + k * WB, WB)

        @pl.when(s < NS - 1)
        def _():
            wb_chunk(my_row0 + WB_FULL * WB, WB_TAIL)

        @pl.when(s == NS - 1)
        def _():
            wb_chunk(my_row0 + WB_FULL * WB, WB_TAIL_LAST)

    return layer


_layer_mid = _make_layer(1.0)
_layer_last = _make_layer(0.25)


def kernel(user_emb, item_emb, edge_index, edge_vals):
    ego = jnp.concatenate([user_emb, item_emb], axis=0)
    egoh = jnp.stack([ego[:, :DH], ego[:, DH:]], axis=0)
    edata = edge_index.reshape(2, NB, BLK).transpose(1, 0, 2)
    e1, s1 = _layer_mid(egoh, edata, edge_vals, egoh)
    e2, s2 = _layer_mid(e1, edata, edge_vals, s1)
    _, s3 = _layer_last(e2, edata, edge_vals, s2)
    mean = jnp.concatenate([s3[0], s3[1]], axis=1)
    return (mean[:USER_N], mean[USER_N:])
